# baseline (speedup 1.0000x reference)
import jax
import jax.numpy as jnp
from jax.experimental import pallas as pl
from jax.experimental.pallas import tpu as pltpu
from jax.experimental.pallas import tpu_sc as plsc

BATCH = 16384
FACTOR = 32
MLP_DIM = 128
GATHER_WINDOW = 256

def _vector_mesh():
    return plsc.VectorSubcoreMesh(
        core_axis_name="core", subcore_axis_name="subcore"
    )


def _gather_pipeline(table_hbm, idx_hbm, out_hbm):
    def body(idx_vmem, out_vmem):
        pltpu.sync_copy(table_hbm.at[idx_vmem.at[0]], out_vmem)

    pltpu.emit_pipeline(
        body,
        grid=(BATCH // GATHER_WINDOW,),
        in_specs=[pl.BlockSpec((1, GATHER_WINDOW), index_map=lambda i: (0, i))],
        out_specs=[pl.BlockSpec((GATHER_WINDOW, 128), index_map=lambda i: (i, 0))],
        core_axis_name=("core", "subcore"),
        dimension_semantics=(pltpu.PARALLEL,),
    )(idx_hbm, out_hbm)


def _sc_gather_mlp(user2, item2, W_user_mlp, W_item_mlp):
    out_types = (
        jax.ShapeDtypeStruct((BATCH, MLP_DIM), jnp.float32),
        jax.ShapeDtypeStruct((BATCH, MLP_DIM), jnp.float32),
    )

    @pl.kernel(out_type=out_types, mesh=_vector_mesh(), scratch_types=[])
    def gather_mlp(u_hbm, i_hbm, wum_hbm, wim_hbm, eum_hbm, eim_hbm):
        _gather_pipeline(wum_hbm, u_hbm, eum_hbm)
        _gather_pipeline(wim_hbm, i_hbm, eim_hbm)

    return gather_mlp(user2, item2, W_user_mlp, W_item_mlp)


def _sc_gather_gmf(u2half, i2half, Wpack):
    out_types = (
        jax.ShapeDtypeStruct((BATCH, 128), jnp.float32),
        jax.ShapeDtypeStruct((BATCH, 128), jnp.float32),
    )

    @pl.kernel(out_type=out_types, mesh=_vector_mesh(), scratch_types=[])
    def gather_gmf(u_hbm, i_hbm, wp_hbm, gu_hbm, gi_hbm):
        _gather_pipeline(wp_hbm, u_hbm, gu_hbm)
        _gather_pipeline(wp_hbm, i_hbm, gi_hbm)

    return gather_gmf(u2half, i2half, Wpack)


def _tc_pack_kernel(wua_ref, wia_ref, wub_ref, wib_ref, eye_ref, out_ref):
    xall = jnp.concatenate(
        [wua_ref[...], wia_ref[...], wub_ref[...], wib_ref[...]], axis=0)
    hi = xall.astype(jnp.bfloat16)
    lo = (xall - hi.astype(jnp.float32)).astype(jnp.bfloat16)
    eye = eye_ref[...]
    dn = (((0,), (0,)), ((), ()))
    out = jax.lax.dot_general(hi, eye, dn, preferred_element_type=jnp.float32)
    out += jax.lax.dot_general(lo, eye, dn, preferred_element_type=jnp.float32)
    out_ref[...] = out


PACK_BLK = 7168
PACK_STEPS = 7
PACK_H = PACK_BLK * PACK_STEPS


def _tc_pack(WugT, WigT):
    eye = jnp.eye(128, dtype=jnp.float32)
    colblk = lambda off: pl.BlockSpec((FACTOR, PACK_BLK),
                                      lambda i, off=off: (0, i + off))
    eye = eye.astype(jnp.bfloat16)
    smallspec = pl.BlockSpec((128, 128), lambda i: (0, 0))
    return pl.pallas_call(
        _tc_pack_kernel,
        grid=(PACK_STEPS,),
        in_specs=[colblk(0), colblk(0), colblk(PACK_STEPS), colblk(PACK_STEPS),
                  smallspec],
        out_specs=pl.BlockSpec((PACK_BLK, 128), lambda i: (i, 0)),
        out_shape=jax.ShapeDtypeStruct((PACK_H, 128), jnp.float32),
    )(WugT, WigT, WugT, WigT, eye)


def _tc_mlp_kernel(eum_ref, eim_ref, w0a_ref, w0b_ref, b0_ref,
                   w1_ref, b1_ref, w2_ref, b2_ref, pwm_ref, out_ref):
    h0 = jnp.dot(eum_ref[...].astype(jnp.bfloat16), w0a_ref[...],
                 preferred_element_type=jnp.float32)
    h0 += jnp.dot(eim_ref[...].astype(jnp.bfloat16), w0b_ref[...],
                  preferred_element_type=jnp.float32)
    h0 = jnp.maximum(h0 + b0_ref[...], 0.0)
    h1 = jnp.dot(h0.astype(jnp.bfloat16), w1_ref[...],
                 preferred_element_type=jnp.float32)
    h1 = jnp.maximum(h1 + b1_ref[...], 0.0)
    h2 = jnp.dot(h1.astype(jnp.bfloat16), w2_ref[...],
                 preferred_element_type=jnp.float32)
    h2 = jnp.maximum(h2 + b2_ref[...], 0.0)
    out_ref[...] = jnp.dot(h2, pwm_ref[...],
                           preferred_element_type=jnp.float32)


def _tc_gmf_kernel(gu_ref, gi_ref, par_ref, mlp_ref, pwg_ref, pb_ref,
                   out_ref):
    gu = gu_ref[...]
    gi = gi_ref[...]
    ub = par_ref[:, 0:1] > 0.5
    ib = par_ref[:, 1:2] > 0.5
    gu_al = jnp.where(ub, jnp.roll(gu, -64, axis=1), gu)
    gi_al = jnp.where(ib, jnp.roll(gi, -96, axis=1),
                      jnp.roll(gi, -32, axis=1))
    g = gu_al * gi_al
    pred = jnp.dot(g, pwg_ref[...], preferred_element_type=jnp.float32)
    out_ref[...] = pred + mlp_ref[...] + pb_ref[0, 0]


def kernel(user, item, W_user_gmf, W_item_gmf, W_user_mlp, W_item_mlp,
           mlp_W0, mlp_b0, mlp_W1, mlp_b1, mlp_W2, mlp_b2, pred_W, pred_b):
    user = user.astype(jnp.int32)
    item = item.astype(jnp.int32)
    user2 = user.reshape(1, BATCH)
    item2 = item.reshape(1, BATCH)

    Wpack = _tc_pack(W_user_gmf.T, W_item_gmf.T)
    upar = (user2 >= PACK_H).astype(jnp.int32)
    ipar = (item2 >= PACK_H).astype(jnp.int32)
    u2half = user2 - upar * PACK_H
    i2half = item2 - ipar * PACK_H
    par = jnp.stack([upar.reshape(BATCH).astype(jnp.float32),
                     ipar.reshape(BATCH).astype(jnp.float32)], axis=1)

    eu_mlp, ei_mlp = _sc_gather_mlp(user2, item2, W_user_mlp, W_item_mlp)
    Wpack, eu_mlp, ei_mlp = jax.lax.optimization_barrier(
        (Wpack, eu_mlp, ei_mlp))
    gu, gi = _sc_gather_gmf(u2half, i2half, Wpack)

    w0a = mlp_W0[:, :MLP_DIM].T.astype(jnp.bfloat16)
    w0b = mlp_W0[:, MLP_DIM:].T.astype(jnp.bfloat16)
    w1 = mlp_W1.T.astype(jnp.bfloat16)
    w2 = mlp_W2.T.astype(jnp.bfloat16)
    b0 = mlp_b0.reshape(1, -1)
    b1 = mlp_b1.reshape(1, -1)
    b2 = mlp_b2.reshape(1, -1)
    pwg = jnp.pad(pred_W[:, :FACTOR], ((0, 0), (0, 128 - FACTOR))).T
    pwm = pred_W[:, FACTOR:].T
    pb = pred_b.reshape(1, 1)

    blk = 4096
    grid = (BATCH // blk,)
    row_spec = lambda d: pl.BlockSpec((blk, d), lambda i: (i, 0))
    full = lambda a: pl.BlockSpec(a.shape, lambda i: (0,) * a.ndim)

    mlp_part = pl.pallas_call(
        _tc_mlp_kernel,
        grid=grid,
        in_specs=[
            row_spec(MLP_DIM), row_spec(MLP_DIM),
            full(w0a), full(w0b), full(b0),
            full(w1), full(b1), full(w2), full(b2), full(pwm),
        ],
        out_specs=pl.BlockSpec((blk, 1), lambda i: (i, 0)),
        out_shape=jax.ShapeDtypeStruct((BATCH, 1), jnp.float32),
    )(eu_mlp, ei_mlp, w0a, w0b, b0, w1, b1, w2, b2, pwm)

    out = pl.pallas_call(
        _tc_gmf_kernel,
        grid=grid,
        in_specs=[
            row_spec(128), row_spec(128), row_spec(2),
            pl.BlockSpec((blk, 1), lambda i: (i, 0)),
            full(pwg), full(pb),
        ],
        out_specs=pl.BlockSpec((blk, 1), lambda i: (i, 0)),
        out_shape=jax.ShapeDtypeStruct((BATCH, 1), jnp.float32),
    )(gu, gi, par, mlp_part, pwg, pb)
    return out.reshape(-1)


# speedup vs baseline: 1.0631x; 1.0631x over previous
import jax
import jax.numpy as jnp
from jax.experimental import pallas as pl
from jax.experimental.pallas import tpu as pltpu
from jax.experimental.pallas import tpu_sc as plsc

BATCH = 16384
FACTOR = 32
MLP_DIM = 128
GATHER_WINDOW = 256

def _vector_mesh():
    return plsc.VectorSubcoreMesh(
        core_axis_name="core", subcore_axis_name="subcore"
    )


def _gather_pipeline(table_hbm, idx_hbm, out_hbm):
    def body(idx_vmem, out_vmem):
        pltpu.sync_copy(table_hbm.at[idx_vmem.at[0]], out_vmem)

    pltpu.emit_pipeline(
        body,
        grid=(BATCH // GATHER_WINDOW,),
        in_specs=[pl.BlockSpec((1, GATHER_WINDOW), index_map=lambda i: (0, i))],
        out_specs=[pl.BlockSpec((GATHER_WINDOW, 128), index_map=lambda i: (i, 0))],
        core_axis_name=("core", "subcore"),
        dimension_semantics=(pltpu.PARALLEL,),
    )(idx_hbm, out_hbm)


def _sc_gather_mlp(user2, item2, W_user_mlp, W_item_mlp):
    out_types = (
        jax.ShapeDtypeStruct((BATCH, MLP_DIM), jnp.float32),
        jax.ShapeDtypeStruct((BATCH, MLP_DIM), jnp.float32),
    )

    @pl.kernel(out_type=out_types, mesh=_vector_mesh(), scratch_types=[])
    def gather_mlp(u_hbm, i_hbm, wum_hbm, wim_hbm, eum_hbm, eim_hbm):
        _gather_pipeline(wum_hbm, u_hbm, eum_hbm)
        _gather_pipeline(wim_hbm, i_hbm, eim_hbm)

    return gather_mlp(user2, item2, W_user_mlp, W_item_mlp)


def _sc_gather_gmf(u2half, i2half, Wpack):
    out_types = (
        jax.ShapeDtypeStruct((BATCH, 128), jnp.float32),
        jax.ShapeDtypeStruct((BATCH, 128), jnp.float32),
    )

    @pl.kernel(out_type=out_types, mesh=_vector_mesh(), scratch_types=[])
    def gather_gmf(u_hbm, i_hbm, wp_hbm, gu_hbm, gi_hbm):
        _gather_pipeline(wp_hbm, u_hbm, gu_hbm)
        _gather_pipeline(wp_hbm, i_hbm, gi_hbm)

    return gather_gmf(u2half, i2half, Wpack)


def _tc_pack_kernel(wua_ref, wia_ref, wub_ref, wib_ref, eye_ref, out_ref):
    xall = jnp.concatenate(
        [wua_ref[...], wia_ref[...], wub_ref[...], wib_ref[...]], axis=0)
    out_ref[...] = jax.lax.dot_general(
        xall, eye_ref[...], (((0,), (0,)), ((), ())),
        preferred_element_type=jnp.float32)


PACK_BLK = 3584
PACK_STEPS = 14
PACK_H = PACK_BLK * PACK_STEPS


def _tc_pack(WugT, WigT):
    eye = jnp.eye(128, dtype=jnp.float32)
    colblk = lambda off: pl.BlockSpec((FACTOR, PACK_BLK),
                                      lambda i, off=off: (0, i + off))
    smallspec = pl.BlockSpec((128, 128), lambda i: (0, 0))
    return pl.pallas_call(
        _tc_pack_kernel,
        grid=(PACK_STEPS,),
        in_specs=[colblk(0), colblk(0), colblk(PACK_STEPS), colblk(PACK_STEPS),
                  smallspec],
        out_specs=pl.BlockSpec((PACK_BLK, 128), lambda i: (i, 0)),
        out_shape=jax.ShapeDtypeStruct((PACK_H, 128), jnp.float32),
    )(WugT, WigT, WugT, WigT, eye)


def _tc_dense_kernel(gu_ref, gi_ref, par_ref, eum_ref, eim_ref,
                     w0a_ref, w0b_ref, b0_ref, w1_ref, b1_ref,
                     w2_ref, b2_ref, pwg_ref, pwm_ref, pb_ref, out_ref):
    h0 = jnp.dot(eum_ref[...], w0a_ref[...], preferred_element_type=jnp.float32)
    h0 += jnp.dot(eim_ref[...], w0b_ref[...], preferred_element_type=jnp.float32)
    h0 = jnp.maximum(h0 + b0_ref[...], 0.0)
    h1 = jnp.dot(h0, w1_ref[...], preferred_element_type=jnp.float32)
    h1 = jnp.maximum(h1 + b1_ref[...], 0.0)
    h2 = jnp.dot(h1, w2_ref[...], preferred_element_type=jnp.float32)
    h2 = jnp.maximum(h2 + b2_ref[...], 0.0)

    gu = gu_ref[...]
    gi = gi_ref[...]
    ub = par_ref[:, 0:1] > 0.5
    ib = par_ref[:, 1:2] > 0.5
    gu_al = jnp.where(ub, jnp.roll(gu, -64, axis=1), gu)
    gi_al = jnp.where(ib, jnp.roll(gi, -96, axis=1),
                      jnp.roll(gi, -32, axis=1))
    g = gu_al * gi_al

    pred = jnp.dot(g, pwg_ref[...], preferred_element_type=jnp.float32)
    pred += jnp.dot(h2, pwm_ref[...], preferred_element_type=jnp.float32)
    out_ref[...] = pred + pb_ref[0, 0]


def kernel(user, item, W_user_gmf, W_item_gmf, W_user_mlp, W_item_mlp,
           mlp_W0, mlp_b0, mlp_W1, mlp_b1, mlp_W2, mlp_b2, pred_W, pred_b):
    user = user.astype(jnp.int32)
    item = item.astype(jnp.int32)
    user2 = user.reshape(1, BATCH)
    item2 = item.reshape(1, BATCH)

    Wpack = _tc_pack(W_user_gmf.T, W_item_gmf.T)
    upar = (user2 >= PACK_H).astype(jnp.int32)
    ipar = (item2 >= PACK_H).astype(jnp.int32)
    u2half = user2 - upar * PACK_H
    i2half = item2 - ipar * PACK_H
    par = jnp.stack([upar.reshape(BATCH).astype(jnp.float32),
                     ipar.reshape(BATCH).astype(jnp.float32)], axis=1)

    eu_mlp, ei_mlp = _sc_gather_mlp(user2, item2, W_user_mlp, W_item_mlp)
    Wpack, eu_mlp, ei_mlp = jax.lax.optimization_barrier(
        (Wpack, eu_mlp, ei_mlp))
    gu, gi = _sc_gather_gmf(u2half, i2half, Wpack)

    w0a = mlp_W0[:, :MLP_DIM].T
    w0b = mlp_W0[:, MLP_DIM:].T
    w1 = mlp_W1.T
    w2 = mlp_W2.T
    b0 = mlp_b0.reshape(1, -1)
    b1 = mlp_b1.reshape(1, -1)
    b2 = mlp_b2.reshape(1, -1)
    pwg = jnp.pad(pred_W[:, :FACTOR], ((0, 0), (0, 128 - FACTOR))).T
    pwm = pred_W[:, FACTOR:].T
    pb = pred_b.reshape(1, 1)

    blk = 4096
    grid = (BATCH // blk,)
    row_spec = lambda d: pl.BlockSpec((blk, d), lambda i: (i, 0))
    full = lambda a: pl.BlockSpec(a.shape, lambda i: (0,) * a.ndim)

    out = pl.pallas_call(
        _tc_dense_kernel,
        grid=grid,
        in_specs=[
            row_spec(128), row_spec(128), row_spec(2),
            row_spec(MLP_DIM), row_spec(MLP_DIM),
            full(w0a), full(w0b), full(b0),
            full(w1), full(b1), full(w2), full(b2),
            full(pwg), full(pwm), full(pb),
        ],
        out_specs=pl.BlockSpec((blk, 1), lambda i: (i, 0)),
        out_shape=jax.ShapeDtypeStruct((BATCH, 1), jnp.float32),
    )(gu, gi, par, eu_mlp, ei_mlp,
      w0a, w0b, b0, w1, b1, w2, b2, pwg, pwm, pb)
    return out.reshape(-1)
